# baseline (speedup 1.0000x reference)
import jax
import jax.numpy as jnp
from jax.experimental import pallas as pl


_BATCH_BLK = 8


def _nn_kernel(xt_ref, ct_ref, out_ref):
    xt = xt_ref[...]
    ct = ct_ref[...]
    bb, dd, tt = xt.shape
    xm2 = xt * (-2.0)
    mm = jnp.concatenate(
        [jax.lax.dot_general(
            xm2[i], ct, (((0,), (0,)), ((), ())),
            preferred_element_type=jnp.float32,
        ) for i in range(bb)], axis=0)
    c_sq = jnp.sum(ct * ct, axis=0)[None, :]
    d = mm + c_sq
    idx = jnp.argmin(d, axis=1).astype(jnp.int32)
    out_ref[...] = idx.reshape(bb, tt)


def kernel(x, centroids):
    batch, tokens, fdim = x.shape
    xt = jnp.swapaxes(x, 1, 2)
    ct = centroids.T
    bb = _BATCH_BLK
    assert batch % bb == 0, (batch, bb)
    nblk = batch // bb
    out = pl.pallas_call(
        _nn_kernel,
        grid=(nblk,),
        in_specs=[
            pl.BlockSpec((bb, fdim, tokens), lambda i: (i, 0, 0)),
            pl.BlockSpec(ct.shape, lambda i: (0, 0)),
        ],
        out_specs=pl.BlockSpec((bb, tokens), lambda i: (i, 0)),
        out_shape=jax.ShapeDtypeStruct((batch, tokens), jnp.int32),
    )(xt, ct)
    return out
